# baseline (speedup 1.0000x reference)
import jax
import jax.numpy as jnp
from jax.experimental import pallas as pl
from jax.experimental.pallas import tpu as pltpu

_NUM_REL = 51
_DIM = 1024
_BATCH = 16384
_BM = 1024

def _body(feat_ref, w_ref, probs_ref):
    logits = jnp.dot(feat_ref[...], w_ref[...], preferred_element_type=jnp.float32)
    m = jnp.max(logits, axis=-1, keepdims=True)
    shifted = logits - m
    e = jnp.exp(shifted)
    s = jnp.sum(e, axis=-1, keepdims=True)
    log_probs = shifted - jnp.log(s)
    probs_ref[...] = jnp.exp(log_probs)

def kernel(feat, labels, W, b):
    probs = pl.pallas_call(
        _body,
        grid=(_BATCH // _BM,),
        in_specs=[
            pl.BlockSpec((_BM, _DIM), lambda i: (i, 0)),
            pl.BlockSpec((_DIM, _NUM_REL), lambda i: (0, 0)),
        ],
        out_specs=pl.BlockSpec((_BM, _NUM_REL), lambda i: (i, 0)),
        out_shape=jax.ShapeDtypeStruct((_BATCH, _NUM_REL), jnp.float32),
    )(feat, W)
    return probs
